# baseline (speedup 1.0000x reference)
import functools

import jax
import jax.numpy as jnp
from jax import lax
from jax.experimental import pallas as pl
from jax.experimental.pallas import tpu as pltpu
from jax.experimental.pallas import tpu_sc as plsc

W = 24
HIDDEN = 768
NUM_POS = W * W + 1
N = 131072

NC = 2
NS = 16
NW = NC * NS
ROWS_PER_WORKER = N // NW
CHUNK = 32
NCHUNKS = ROWS_PER_WORKER // CHUNK
NBUF = 4
NJ = NCHUNKS // NBUF
NREP = 6

_mesh = plsc.VectorSubcoreMesh(core_axis_name="c", subcore_axis_name="s")


@functools.partial(
    pl.kernel,
    mesh=_mesh,
    out_type=jax.ShapeDtypeStruct((N, HIDDEN), jnp.float32),
    scratch_types=[
        pltpu.VMEM((ROWS_PER_WORKER,), jnp.float32),
        pltpu.VMEM((ROWS_PER_WORKER,), jnp.float32),
    ] + [pltpu.VMEM((CHUNK,), jnp.int32) for _ in range(NBUF)]
      + [pltpu.VMEM((CHUNK, HIDDEN), jnp.float32) for _ in range(NBUF)]
      + [pltpu.SemaphoreType.DMA for _ in range(2 * NBUF)],
)
def _pos_embed_gather(pos_hbm, table_hbm, out_hbm, xbuf, ybuf, *bufs):
    idxb = bufs[:NBUF]
    rowb = bufs[NBUF:2 * NBUF]
    gsem = bufs[2 * NBUF:3 * NBUF]
    osem = bufs[3 * NBUF:]

    cid = lax.axis_index("c")
    sid = lax.axis_index("s")
    wid = sid * NC + cid
    base = wid * ROWS_PER_WORKER
    rep_off = (wid % NREP) * NUM_POS

    pltpu.sync_copy(pos_hbm.at[0, pl.ds(base, ROWS_PER_WORKER)], xbuf)
    pltpu.sync_copy(pos_hbm.at[1, pl.ds(base, ROWS_PER_WORKER)], ybuf)

    def compute_idx(ci, b):
        def idx_body(i, c):
            x = xbuf[pl.ds(ci * CHUNK + i * 16, 16)]
            y = ybuf[pl.ds(ci * CHUNK + i * 16, 16)]
            fx = (x * 24.0).astype(jnp.int32)
            fy = (y * 24.0).astype(jnp.int32)
            idxb[b][pl.ds(i * 16, 16)] = fx * 24 + fy + 1 + rep_off
            return c
        lax.fori_loop(0, CHUNK // 16, idx_body, 0, unroll=True)

    def out_slice(ci):
        return out_hbm.at[pl.ds(base + ci * CHUNK, CHUNK)]

    def finish(ci, b):
        pltpu.make_async_copy(table_hbm.at[idxb[b]], rowb[b], gsem[b]).wait()
        pltpu.async_copy(rowb[b], out_slice(ci), osem[b])

    def ring_body(j, carry):
        for b in range(NBUF):
            ci = NBUF * j + b

            @pl.when(j > 0)
            def _reclaim():
                pltpu.make_async_copy(rowb[b], out_slice(ci - NBUF), osem[b]).wait()

            compute_idx(ci, b)
            pltpu.async_copy(table_hbm.at[idxb[b]], rowb[b], gsem[b])

            pb = (b + 1) % NBUF
            pci = ci - (NBUF - 1)
            if b == NBUF - 1:
                finish(pci, pb)
            else:
                @pl.when(j > 0)
                def _fin():
                    finish(pci, pb)
        return carry

    lax.fori_loop(0, NJ, ring_body, 0)

    for t in range(NBUF - 1):
        pci = NCHUNKS - (NBUF - 1) + t
        finish(pci, pci % NBUF)
    for b in range(NBUF):
        pltpu.make_async_copy(
            rowb[b], out_slice(NCHUNKS - NBUF + b), osem[b]).wait()


def kernel(pos, positional_embeddings):
    table = positional_embeddings.reshape(NUM_POS, HIDDEN)
    table_rep = jnp.broadcast_to(table, (NREP, NUM_POS, HIDDEN))
    table_rep = table_rep.reshape(NREP * NUM_POS, HIDDEN)
    out = _pos_embed_gather(pos.T, table_rep)
    return out.reshape(1, N, HIDDEN)


# speedup vs baseline: 1.0212x; 1.0212x over previous
import functools

import jax
import jax.numpy as jnp
from jax import lax
from jax.experimental import pallas as pl
from jax.experimental.pallas import tpu as pltpu
from jax.experimental.pallas import tpu_sc as plsc

W = 24
HIDDEN = 768
NUM_POS = W * W + 1
N = 131072

NC = 2
NS = 16
NW = NC * NS
ROWS_PER_WORKER = N // NW
CHUNK = 32
NCHUNKS = ROWS_PER_WORKER // CHUNK
NBUF = 4
NJ = NCHUNKS // NBUF
NREP = 4

_mesh = plsc.VectorSubcoreMesh(core_axis_name="c", subcore_axis_name="s")


@functools.partial(
    pl.kernel,
    mesh=_mesh,
    out_type=jax.ShapeDtypeStruct((N, HIDDEN), jnp.float32),
    scratch_types=[
        pltpu.VMEM((ROWS_PER_WORKER,), jnp.float32),
        pltpu.VMEM((ROWS_PER_WORKER,), jnp.float32),
    ] + [pltpu.VMEM((CHUNK,), jnp.int32) for _ in range(NBUF)]
      + [pltpu.VMEM((CHUNK, HIDDEN), jnp.float32) for _ in range(NBUF)]
      + [pltpu.SemaphoreType.DMA for _ in range(2 * NBUF)],
)
def _pos_embed_gather(pos_hbm, table_hbm, out_hbm, xbuf, ybuf, *bufs):
    idxb = bufs[:NBUF]
    rowb = bufs[NBUF:2 * NBUF]
    gsem = bufs[2 * NBUF:3 * NBUF]
    osem = bufs[3 * NBUF:]

    cid = lax.axis_index("c")
    sid = lax.axis_index("s")
    wid = sid * NC + cid
    base = wid * ROWS_PER_WORKER
    rep_off = (wid % NREP) * NUM_POS

    pltpu.sync_copy(pos_hbm.at[0, pl.ds(base, ROWS_PER_WORKER)], xbuf)
    pltpu.sync_copy(pos_hbm.at[1, pl.ds(base, ROWS_PER_WORKER)], ybuf)

    def compute_idx(ci, b):
        def idx_body(i, c):
            x = xbuf[pl.ds(ci * CHUNK + i * 16, 16)]
            y = ybuf[pl.ds(ci * CHUNK + i * 16, 16)]
            fx = (x * 24.0).astype(jnp.int32)
            fy = (y * 24.0).astype(jnp.int32)
            idxb[b][pl.ds(i * 16, 16)] = fx * 24 + fy + 1 + rep_off
            return c
        lax.fori_loop(0, CHUNK // 16, idx_body, 0, unroll=True)

    def out_slice(ci):
        return out_hbm.at[pl.ds(base + ci * CHUNK, CHUNK)]

    def finish(ci, b):
        pltpu.make_async_copy(table_hbm.at[idxb[b]], rowb[b], gsem[b]).wait()
        pltpu.async_copy(rowb[b], out_slice(ci), osem[b])

    def ring_body(j, carry):
        for b in range(NBUF):
            ci = NBUF * j + b

            @pl.when(j > 0)
            def _reclaim():
                pltpu.make_async_copy(rowb[b], out_slice(ci - NBUF), osem[b]).wait()

            compute_idx(ci, b)
            pltpu.async_copy(table_hbm.at[idxb[b]], rowb[b], gsem[b])

            pb = (b + 1) % NBUF
            pci = ci - (NBUF - 1)
            if b == NBUF - 1:
                finish(pci, pb)
            else:
                @pl.when(j > 0)
                def _fin():
                    finish(pci, pb)
        return carry

    lax.fori_loop(0, NJ, ring_body, 0)

    for t in range(NBUF - 1):
        pci = NCHUNKS - (NBUF - 1) + t
        finish(pci, pci % NBUF)
    for b in range(NBUF):
        pltpu.make_async_copy(
            rowb[b], out_slice(NCHUNKS - NBUF + b), osem[b]).wait()


def kernel(pos, positional_embeddings):
    table = positional_embeddings.reshape(NUM_POS, HIDDEN)
    table_rep = jnp.broadcast_to(table, (NREP, NUM_POS, HIDDEN))
    table_rep = table_rep.reshape(NREP * NUM_POS, HIDDEN)
    out = _pos_embed_gather(pos.T, table_rep)
    return out.reshape(1, N, HIDDEN)
